# baseline (speedup 1.0000x reference)
import jax
import jax.numpy as jnp
from jax.experimental import pallas as pl

N_NODES = 10000
IN_FEAT = 128
OUT_FEAT = 128
NUM_RELS = 16
NUM_BASES = 4

ROW_TILE = 400


def _mm_body(agg_ref, feat_ref, wb_ref, ws_ref, bias_ref, out_ref):
    acc = jnp.dot(agg_ref[...], wb_ref[...], preferred_element_type=jnp.float32)
    acc += jnp.dot(feat_ref[...], ws_ref[...], preferred_element_type=jnp.float32)
    out_ref[...] = acc + bias_ref[...]


def _project(agg_flat, feat, W, bias):
    n = feat.shape[0]
    wb = W[:NUM_BASES].reshape(NUM_BASES * IN_FEAT, OUT_FEAT)
    ws = W[NUM_BASES]
    grid = (n // ROW_TILE,)
    return pl.pallas_call(
        _mm_body,
        grid=grid,
        in_specs=[
            pl.BlockSpec((ROW_TILE, NUM_BASES * IN_FEAT), lambda i: (i, 0)),
            pl.BlockSpec((ROW_TILE, IN_FEAT), lambda i: (i, 0)),
            pl.BlockSpec((NUM_BASES * IN_FEAT, OUT_FEAT), lambda i: (0, 0)),
            pl.BlockSpec((IN_FEAT, OUT_FEAT), lambda i: (0, 0)),
            pl.BlockSpec((OUT_FEAT,), lambda i: (0,)),
        ],
        out_specs=pl.BlockSpec((ROW_TILE, OUT_FEAT), lambda i: (i, 0)),
        out_shape=jax.ShapeDtypeStruct((n, OUT_FEAT), jnp.float32),
    )(agg_flat, feat, wb, ws, bias)


def kernel(feat, edge_index, etypes, W, coeff, bias):
    src = edge_index[0]
    dst = edge_index[1]
    gathered = jnp.take(feat, src, axis=0)
    ecoef = jnp.take(coeff, etypes, axis=0)
    weighted = gathered[:, None, :] * ecoef[:, :, None]
    agg = jnp.zeros((N_NODES, NUM_BASES, IN_FEAT), dtype=feat.dtype).at[dst].add(weighted)
    agg_flat = agg.reshape(N_NODES, NUM_BASES * IN_FEAT)
    return _project(agg_flat, feat, W, bias)


# speedup vs baseline: 53.2852x; 53.2852x over previous
import functools

import jax
import jax.numpy as jnp
from jax import lax
from jax.experimental import pallas as pl
from jax.experimental.pallas import tpu as pltpu
from jax.experimental.pallas import tpu_sc as plsc

N_NODES = 10000
E_EDGES = 320000
D = 128
NB = 4
NR = 16
NC = 2
NS = 16
NW = NC * NS
L = 16

NPASS = 5
NRANGE = NW * NPASS
NLOC = 64
NBINS = NRANGE
AGG_WORDS = NLOC * NB * D

E_SLAB = E_EDGES // NW
EC = 2000
CCAP = 128
OVF_CAP = 10240
OG = EC // L

GB = 128
CAP = 8192

ROW_TILE = 400

_MESH = dict(core_axis_name="c", subcore_axis_name="s")


def _sc_bin(packed):
    @functools.partial(
        pl.kernel,
        out_type=(
            jax.ShapeDtypeStruct((NW * NBINS * CCAP,), jnp.int32),
            jax.ShapeDtypeStruct((NW * OVF_CAP,), jnp.int32),
            jax.ShapeDtypeStruct((NW * L,), jnp.int32),
        ),
        mesh=plsc.VectorSubcoreMesh(**_MESH),
        scratch_types=[
            pltpu.VMEM((NBINS * CCAP,), jnp.int32),
            pltpu.VMEM((NBINS + L,), jnp.int32),
            pltpu.VMEM((OVF_CAP + L,), jnp.int32),
            pltpu.VMEM((L,), jnp.int32),
            pltpu.VMEM((EC,), jnp.int32),
        ],
        compiler_params=pltpu.CompilerParams(needs_layout_passes=False),
    )
    def bin_kernel(packed_hbm, binned_hbm, ovf_hbm, ovfcnt_hbm,
                   cells, cnt, ovfb, vtmp, pbuf):
        cidx = lax.axis_index("c")
        sidx = lax.axis_index("s")
        wid = sidx * NC + cidx

        sent = jnp.full((L,), -1, jnp.int32)
        zv = jnp.zeros((L,), jnp.int32)

        @plsc.parallel_loop(0, NBINS * CCAP // L, unroll=8)
        def _fill(i):
            cells[pl.ds(i * L, L)] = sent

        @plsc.parallel_loop(0, (NBINS + L) // L, unroll=1)
        def _zcnt(i):
            cnt[pl.ds(i * L, L)] = zv

        def wbody(g, ocur):
            v = pbuf[pl.ds(g * L, L)]
            bins = (v >> 24) & 0xFF
            rank, last = plsc.scan_count(bins)
            cvals = plsc.load_gather(cnt, [bins])
            pos = cvals + rank - 1
            ok = pos < CCAP
            plsc.store_scatter(cells, [bins * CCAP + pos], v, mask=ok)
            plsc.store_scatter(cnt, [bins], cvals + rank, mask=last)
            movf = jnp.logical_not(ok)
            opos = ocur + plsc.cumsum(movf.astype(jnp.int32)) - 1
            plsc.store_scatter(ovfb, [opos], v, mask=movf)
            return ocur + plsc.all_reduce_population_count(movf)

        ocur = jnp.zeros((L,), jnp.int32)
        for c in range(E_SLAB // EC):
            pltpu.sync_copy(
                packed_hbm.at[pl.ds(wid * E_SLAB + c * EC, EC)], pbuf)
            ocur = lax.fori_loop(0, EC // L, wbody, ocur)

        plsc.store_scatter(ovfb, [ocur + lax.iota(jnp.int32, L)], sent)
        vtmp[...] = ocur
        pltpu.sync_copy(vtmp, ovfcnt_hbm.at[pl.ds(wid * L, L)])
        pltpu.sync_copy(cells, binned_hbm.at[pl.ds(wid * NBINS * CCAP, NBINS * CCAP)])
        pltpu.sync_copy(ovfb.at[pl.ds(0, OVF_CAP)],
                        ovf_hbm.at[pl.ds(wid * OVF_CAP, OVF_CAP)])

    return bin_kernel(packed)


def _sc_aggregate(binned, ovf, ovfcnt, feat, coeffr):
    @functools.partial(
        pl.kernel,
        out_type=jax.ShapeDtypeStruct((NRANGE * NLOC * NB * D,), jnp.float32),
        mesh=plsc.VectorSubcoreMesh(**_MESH),
        scratch_types=[
            pltpu.VMEM((CAP + L,), jnp.int32),
            pltpu.VMEM((GB,), jnp.int32),
            pltpu.VMEM((GB,), jnp.int32),
            pltpu.VMEM((2 * GB, D), jnp.float32),
            pltpu.VMEM((AGG_WORDS,), jnp.float32),
            pltpu.VMEM((NW * CCAP,), jnp.int32),
            pltpu.VMEM((EC,), jnp.int32),
            pltpu.VMEM((NW * L,), jnp.int32),
            pltpu.VMEM((NR * NB + L,), jnp.float32),
            pltpu.SemaphoreType.DMA,
            pltpu.SemaphoreType.DMA,
            pltpu.SemaphoreType.DMA,
        ],
        compiler_params=pltpu.CompilerParams(needs_layout_passes=False),
    )
    def agg_kernel(binned_hbm, ovf_hbm, ovfcnt_hbm, feat_hbm, coeff_hbm, out_hbm,
                   psel, gidxA, gidxB, rows, agg, cellbuf, pbuf, ocbuf, coeffv,
                   csem, gsemA, gsemB):
        cidx = lax.axis_index("c")
        sidx = lax.axis_index("s")
        wid = sidx * NC + cidx
        pltpu.sync_copy(coeff_hbm, coeffv.at[pl.ds(0, NR * NB)])
        pltpu.sync_copy(ovfcnt_hbm, ocbuf)

        def accumulate(off, count, rbase, base):
            def acc(e):
                pv = psel[pl.ds(off + e, L)][0]
                dl = ((pv >> 18) & 0x3FFF) - base
                et = (pv >> 14) & 0xF
                cv = coeffv[pl.ds(et * NB, L)]
                c0 = cv[0]
                c1 = cv[1]
                c2 = cv[2]
                c3 = cv[3]
                aoff = dl * (NB * D)
                fs = [rows[rbase + e, pl.ds(q * L, L)] for q in range(D // L)]
                for q in range(D // L):
                    f = fs[q]
                    plsc.addupdate(agg.at[pl.ds(aoff + 0 * D + q * L, L)], f * c0)
                    plsc.addupdate(agg.at[pl.ds(aoff + 1 * D + q * L, L)], f * c1)
                    plsc.addupdate(agg.at[pl.ds(aoff + 2 * D + q * L, L)], f * c2)
                    plsc.addupdate(agg.at[pl.ds(aoff + 3 * D + q * L, L)], f * c3)
            unroll = 2 if isinstance(count, int) else 1
            plsc.parallel_loop(0, count, unroll=unroll)(acc)

        def pass_body(p, _carry):
            rid = p * NW + wid
            base = rid * NLOC

            zvf = jnp.zeros((L,), jnp.float32)

            @plsc.parallel_loop(0, AGG_WORDS // L, unroll=8)
            def _zero(i):
                agg[pl.ds(i * L, L)] = zvf

            def cstart(w2, _):
                pltpu.make_async_copy(
                    binned_hbm.at[pl.ds(w2 * NBINS * CCAP + rid * CCAP, CCAP)],
                    cellbuf.at[pl.ds(w2 * CCAP, CCAP)], csem).start()
                return 0
            lax.fori_loop(0, NW, cstart, 0)

            def cwait(w2, _):
                pltpu.make_async_copy(
                    binned_hbm.at[pl.ds(rid * CCAP, CCAP)],
                    cellbuf.at[pl.ds(0, CCAP)], csem).wait()
                return 0
            lax.fori_loop(0, NW, cwait, 0)

            def build_idx(gidx, off):
                for q in range(GB // L):
                    pv = psel[pl.ds(off + q * L, L)]
                    gidx[pl.ds(q * L, L)] = pv & 0x3FFF

            rowsA = rows.at[pl.ds(0, GB)]
            rowsB = rows.at[pl.ds(GB, GB)]

            def flush_pair(np_):
                offA = np_ & (CAP - 1)
                offB = (np_ + GB) & (CAP - 1)
                build_idx(gidxA, offA)
                gdA = pltpu.make_async_copy(feat_hbm.at[gidxA], rowsA, gsemA)
                gdA.start()
                build_idx(gidxB, offB)
                gdB = pltpu.make_async_copy(feat_hbm.at[gidxB], rowsB, gsemB)
                gdB.start()
                gdA.wait()
                accumulate(offA, GB, 0, base)
                gdB.wait()
                accumulate(offB, GB, GB, base)
                return np_ + 2 * GB

            def flush_batch(np_):
                off = np_ & (CAP - 1)
                build_idx(gidxA, off)
                gd = pltpu.make_async_copy(feat_hbm.at[gidxA], rowsA, gsemA)
                gd.start()
                gd.wait()
                accumulate(off, GB, 0, base)
                return np_ + GB

            def scan_g(buf, g, cv):
                v = buf[pl.ds(g * L, L)]
                dstf = (v >> 18) & 0x3FFF
                m = (dstf >= base) & (dstf < base + NLOC)
                mi = m.astype(jnp.int32)
                pos = cv + plsc.cumsum(mi) - 1
                plsc.store_scatter(psel, [pos & (CAP - 1)], v, mask=m)
                return cv + plsc.all_reduce_population_count(m)

            cur_vec = plsc.parallel_loop(
                0, NW * CCAP // L, unroll=5,
                carry=jnp.zeros((L,), jnp.int32))(
                    lambda g, cv: scan_g(cellbuf, g, cv))

            curs = jnp.max(cur_vec)
            nproc = lax.while_loop(
                lambda np_: np_ + 2 * GB <= curs, flush_pair, jnp.int32(0))
            nproc = lax.while_loop(
                lambda np_: np_ + GB <= curs, flush_batch, nproc)

            def ovf_w(w2, carry):
                cur_vec2, nproc2 = carry
                cw = ocbuf[pl.ds(w2 * L, L)][0]
                ngroups = (cw + L - 1) >> 4
                nchunks = (ngroups + OG - 1) // OG

                def ochunk(c, carry3):
                    cur_vec3, nproc3 = carry3
                    pltpu.sync_copy(
                        ovf_hbm.at[pl.ds(w2 * OVF_CAP + c * EC, EC)], pbuf)
                    gs = jnp.minimum(ngroups - c * OG, OG)
                    cur_vec3 = lax.fori_loop(
                        0, gs, lambda g, cv: scan_g(pbuf, g, cv), cur_vec3)
                    curs3 = jnp.max(cur_vec3)
                    nproc3 = lax.while_loop(
                        lambda np_: np_ + GB <= curs3, flush_batch, nproc3)
                    return cur_vec3, nproc3

                return lax.fori_loop(0, nchunks, ochunk, (cur_vec2, nproc2))

            cur_vec, nproc = lax.fori_loop(0, NW, ovf_w, (cur_vec, nproc))

            pending = jnp.max(cur_vec) - nproc
            off = nproc & (CAP - 1)
            zidx = jnp.zeros((L,), jnp.int32)
            for q in range(GB // L):
                pv = psel[pl.ds(off + q * L, L)]
                lane = lax.iota(jnp.int32, L) + q * L
                gidxA[pl.ds(q * L, L)] = jnp.where(lane < pending, pv & 0x3FFF, zidx)
            gd = pltpu.make_async_copy(feat_hbm.at[gidxA], rowsA, gsemA)
            gd.start()
            gd.wait()
            accumulate(off, pending, 0, base)

            pltpu.sync_copy(agg, out_hbm.at[pl.ds(base * NB * D, AGG_WORDS)])
            return 0

        lax.fori_loop(0, NPASS, pass_body, 0)

    return agg_kernel(binned, ovf, ovfcnt, feat, coeffr)


def _mm_body(agg_ref, feat_ref, wb_ref, ws_ref, bias_ref, out_ref):
    acc = jnp.dot(agg_ref[...], wb_ref[...], preferred_element_type=jnp.float32)
    acc += jnp.dot(feat_ref[...], ws_ref[...], preferred_element_type=jnp.float32)
    out_ref[...] = acc + bias_ref[...]


def _project(agg_flat, feat, W, bias):
    n = feat.shape[0]
    wb = W[:NB].reshape(NB * D, D)
    ws = W[NB]
    return pl.pallas_call(
        _mm_body,
        grid=(n // ROW_TILE,),
        in_specs=[
            pl.BlockSpec((ROW_TILE, NB * D), lambda i: (i, 0)),
            pl.BlockSpec((ROW_TILE, D), lambda i: (i, 0)),
            pl.BlockSpec((NB * D, D), lambda i: (0, 0)),
            pl.BlockSpec((D, D), lambda i: (0, 0)),
            pl.BlockSpec((D,), lambda i: (0,)),
        ],
        out_specs=pl.BlockSpec((ROW_TILE, D), lambda i: (i, 0)),
        out_shape=jax.ShapeDtypeStruct((n, D), jnp.float32),
    )(agg_flat, feat, wb, ws, bias)


def kernel(feat, edge_index, etypes, W, coeff, bias):
    src = edge_index[0]
    dst = edge_index[1]
    packed = src | (etypes << 14) | (dst << 18)
    binned, ovf, ovfcnt = _sc_bin(packed)
    aggf = _sc_aggregate(binned, ovf, ovfcnt, feat, coeff.reshape(-1))
    agg = aggf.reshape(NRANGE * NLOC, NB * D)[:N_NODES]
    return _project(agg, feat, W, bias)


# speedup vs baseline: 55.9946x; 1.0508x over previous
import functools

import jax
import jax.numpy as jnp
from jax import lax
from jax.experimental import pallas as pl
from jax.experimental.pallas import tpu as pltpu
from jax.experimental.pallas import tpu_sc as plsc

N_NODES = 10000
E_EDGES = 320000
D = 128
NB = 4
NR = 16
NC = 2
NS = 16
NW = NC * NS
L = 16

NPASS = 5
NRANGE = NW * NPASS
NLOC = 64
NBINS = NRANGE
AGG_WORDS = NLOC * NB * D

E_SLAB = E_EDGES // NW
EC = 2000
CCAP = 128
OVF_CAP = 10240
OG = EC // L

GB = 128
CAP = 8192

ROW_TILE = 400

_MESH = dict(core_axis_name="c", subcore_axis_name="s")


def _sc_bin(packed):
    @functools.partial(
        pl.kernel,
        out_type=(
            jax.ShapeDtypeStruct((NW * NBINS * CCAP,), jnp.int32),
            jax.ShapeDtypeStruct((NW * OVF_CAP,), jnp.int32),
            jax.ShapeDtypeStruct((NW * L,), jnp.int32),
        ),
        mesh=plsc.VectorSubcoreMesh(**_MESH),
        scratch_types=[
            pltpu.VMEM((NBINS * CCAP,), jnp.int32),
            pltpu.VMEM((NBINS + L,), jnp.int32),
            pltpu.VMEM((OVF_CAP + L,), jnp.int32),
            pltpu.VMEM((L,), jnp.int32),
            pltpu.VMEM((EC,), jnp.int32),
        ],
        compiler_params=pltpu.CompilerParams(needs_layout_passes=False),
    )
    def bin_kernel(packed_hbm, binned_hbm, ovf_hbm, ovfcnt_hbm,
                   cells, cnt, ovfb, vtmp, pbuf):
        cidx = lax.axis_index("c")
        sidx = lax.axis_index("s")
        wid = sidx * NC + cidx

        sent = jnp.full((L,), -1, jnp.int32)
        zv = jnp.zeros((L,), jnp.int32)

        @plsc.parallel_loop(0, NBINS * CCAP // L, unroll=8)
        def _fill(i):
            cells[pl.ds(i * L, L)] = sent

        @plsc.parallel_loop(0, (NBINS + L) // L, unroll=1)
        def _zcnt(i):
            cnt[pl.ds(i * L, L)] = zv

        def wbody(g, ocur):
            v = pbuf[pl.ds(g * L, L)]
            bins = (v >> 24) & 0xFF
            rank, last = plsc.scan_count(bins)
            cvals = plsc.load_gather(cnt, [bins])
            pos = cvals + rank - 1
            ok = pos < CCAP
            plsc.store_scatter(cells, [bins * CCAP + pos], v, mask=ok)
            plsc.store_scatter(cnt, [bins], cvals + rank, mask=last)
            movf = jnp.logical_not(ok)
            opos = ocur + plsc.cumsum(movf.astype(jnp.int32)) - 1
            plsc.store_scatter(ovfb, [opos], v, mask=movf)
            return ocur + plsc.all_reduce_population_count(movf)

        ocur = jnp.zeros((L,), jnp.int32)
        for c in range(E_SLAB // EC):
            pltpu.sync_copy(
                packed_hbm.at[pl.ds(wid * E_SLAB + c * EC, EC)], pbuf)
            ocur = lax.fori_loop(0, EC // L, wbody, ocur)

        plsc.store_scatter(ovfb, [ocur + lax.iota(jnp.int32, L)], sent)
        vtmp[...] = ocur
        pltpu.sync_copy(vtmp, ovfcnt_hbm.at[pl.ds(wid * L, L)])
        pltpu.sync_copy(cells, binned_hbm.at[pl.ds(wid * NBINS * CCAP, NBINS * CCAP)])
        pltpu.sync_copy(ovfb.at[pl.ds(0, OVF_CAP)],
                        ovf_hbm.at[pl.ds(wid * OVF_CAP, OVF_CAP)])

    return bin_kernel(packed)


def _sc_aggregate(binned, ovf, ovfcnt, feat, coeffr):
    @functools.partial(
        pl.kernel,
        out_type=jax.ShapeDtypeStruct((NRANGE * NLOC * NB * D,), jnp.float32),
        mesh=plsc.VectorSubcoreMesh(**_MESH),
        scratch_types=[
            pltpu.VMEM((CAP + L,), jnp.int32),
            pltpu.VMEM((GB,), jnp.int32),
            pltpu.VMEM((GB,), jnp.int32),
            pltpu.VMEM((2 * GB, D), jnp.float32),
            pltpu.VMEM((AGG_WORDS,), jnp.float32),
            pltpu.VMEM((NW * CCAP,), jnp.int32),
            pltpu.VMEM((EC,), jnp.int32),
            pltpu.VMEM((NW * L,), jnp.int32),
            pltpu.VMEM((NR * NB + L,), jnp.float32),
            pltpu.SemaphoreType.DMA,
            pltpu.SemaphoreType.DMA,
            pltpu.SemaphoreType.DMA,
        ],
        compiler_params=pltpu.CompilerParams(needs_layout_passes=False),
    )
    def agg_kernel(binned_hbm, ovf_hbm, ovfcnt_hbm, feat_hbm, coeff_hbm, out_hbm,
                   psel, gidxA, gidxB, rows, agg, cellbuf, pbuf, ocbuf, coeffv,
                   csem, gsemA, gsemB):
        cidx = lax.axis_index("c")
        sidx = lax.axis_index("s")
        wid = sidx * NC + cidx
        pltpu.sync_copy(coeff_hbm, coeffv.at[pl.ds(0, NR * NB)])
        pltpu.sync_copy(ovfcnt_hbm, ocbuf)

        def accumulate(off, count, rbase, base):
            def acc(e):
                pv = psel[pl.ds(off + e, L)][0]
                dl = ((pv >> 18) & 0x3FFF) - base
                et = (pv >> 14) & 0xF
                cv = coeffv[pl.ds(et * NB, L)]
                c0 = cv[0]
                c1 = cv[1]
                c2 = cv[2]
                c3 = cv[3]
                aoff = dl * (NB * D)
                fs = [rows[rbase + e, pl.ds(q * L, L)] for q in range(D // L)]
                for q in range(D // L):
                    f = fs[q]
                    plsc.addupdate(agg.at[pl.ds(aoff + 0 * D + q * L, L)], f * c0)
                    plsc.addupdate(agg.at[pl.ds(aoff + 1 * D + q * L, L)], f * c1)
                    plsc.addupdate(agg.at[pl.ds(aoff + 2 * D + q * L, L)], f * c2)
                    plsc.addupdate(agg.at[pl.ds(aoff + 3 * D + q * L, L)], f * c3)
            unroll = 2 if isinstance(count, int) else 1
            plsc.parallel_loop(0, count, unroll=unroll)(acc)

        def pass_body(p, _carry):
            rid = p * NW + wid
            base = rid * NLOC

            zvf = jnp.zeros((L,), jnp.float32)

            @plsc.parallel_loop(0, AGG_WORDS // L, unroll=8)
            def _zero(i):
                agg[pl.ds(i * L, L)] = zvf

            def cstart(w2, _):
                pltpu.make_async_copy(
                    binned_hbm.at[pl.ds(w2 * NBINS * CCAP + rid * CCAP, CCAP)],
                    cellbuf.at[pl.ds(w2 * CCAP, CCAP)], csem).start()
                return 0
            lax.fori_loop(0, NW, cstart, 0)

            def cwait(w2, _):
                pltpu.make_async_copy(
                    binned_hbm.at[pl.ds(rid * CCAP, CCAP)],
                    cellbuf.at[pl.ds(0, CCAP)], csem).wait()
                return 0
            lax.fori_loop(0, NW, cwait, 0)

            def build_idx(gidx, off):
                for q in range(GB // L):
                    pv = psel[pl.ds(off + q * L, L)]
                    gidx[pl.ds(q * L, L)] = pv & 0x3FFF

            rowsA = rows.at[pl.ds(0, GB)]
            rowsB = rows.at[pl.ds(GB, GB)]

            def flush_batch(np_):
                off = pl.multiple_of(np_ & (CAP - 1), GB)
                build_idx(gidxA, off)
                gd = pltpu.make_async_copy(feat_hbm.at[gidxA], rowsA, gsemA)
                gd.start()
                gd.wait()
                accumulate(off, GB, 0, base)
                return np_ + GB

            def scan_g(buf, g, cv):
                v = buf[pl.ds(g * L, L)]
                dstf = (v >> 18) & 0x3FFF
                m = (dstf >= base) & (dstf < base + NLOC)
                mi = m.astype(jnp.int32)
                pos = cv + plsc.cumsum(mi) - 1
                plsc.store_scatter(psel, [pos & (CAP - 1)], v, mask=m)
                return cv + plsc.all_reduce_population_count(m)

            cur_vec = plsc.parallel_loop(
                0, NW * CCAP // L, unroll=5,
                carry=jnp.zeros((L,), jnp.int32))(
                    lambda g, cv: scan_g(cellbuf, g, cv))

            curs = jnp.max(cur_vec)
            nbat = curs // GB

            @pl.when(nbat > 0)
            def _():
                build_idx(gidxA, 0)
                pltpu.make_async_copy(feat_hbm.at[gidxA], rowsA, gsemA).start()

            def ring_batch(b, _):
                offn = (b + 1) * GB

                @pl.when((b & 1) == 0)
                def _():
                    @pl.when(b + 1 < nbat)
                    def _():
                        build_idx(gidxB, offn)
                        pltpu.make_async_copy(
                            feat_hbm.at[gidxB], rowsB, gsemB).start()
                    pltpu.make_async_copy(feat_hbm.at[gidxA], rowsA, gsemA).wait()
                    accumulate(b * GB, GB, 0, base)

                @pl.when((b & 1) == 1)
                def _():
                    @pl.when(b + 1 < nbat)
                    def _():
                        build_idx(gidxA, offn)
                        pltpu.make_async_copy(
                            feat_hbm.at[gidxA], rowsA, gsemA).start()
                    pltpu.make_async_copy(feat_hbm.at[gidxB], rowsB, gsemB).wait()
                    accumulate(b * GB, GB, GB, base)
                return 0

            lax.fori_loop(0, nbat, ring_batch, 0)
            nproc = nbat * GB

            def ovf_w(w2, carry):
                cur_vec2, nproc2 = carry
                cw = ocbuf[pl.ds(w2 * L, L)][0]
                ngroups = (cw + L - 1) >> 4
                nchunks = (ngroups + OG - 1) // OG

                def ochunk(c, carry3):
                    cur_vec3, nproc3 = carry3
                    pltpu.sync_copy(
                        ovf_hbm.at[pl.ds(w2 * OVF_CAP + c * EC, EC)], pbuf)
                    gs = jnp.minimum(ngroups - c * OG, OG)
                    cur_vec3 = lax.fori_loop(
                        0, gs, lambda g, cv: scan_g(pbuf, g, cv), cur_vec3)
                    curs3 = jnp.max(cur_vec3)
                    nproc3 = lax.while_loop(
                        lambda np_: np_ + GB <= curs3, flush_batch, nproc3)
                    return cur_vec3, nproc3

                return lax.fori_loop(0, nchunks, ochunk, (cur_vec2, nproc2))

            cur_vec, nproc = lax.fori_loop(0, NW, ovf_w, (cur_vec, nproc))

            pending = jnp.max(cur_vec) - nproc
            off = pl.multiple_of(nproc & (CAP - 1), GB)
            zidx = jnp.zeros((L,), jnp.int32)
            for q in range(GB // L):
                pv = psel[pl.ds(off + q * L, L)]
                lane = lax.iota(jnp.int32, L) + q * L
                gidxA[pl.ds(q * L, L)] = jnp.where(lane < pending, pv & 0x3FFF, zidx)
            gd = pltpu.make_async_copy(feat_hbm.at[gidxA], rowsA, gsemA)
            gd.start()
            gd.wait()
            accumulate(off, pending, 0, base)

            pltpu.sync_copy(agg, out_hbm.at[pl.ds(base * NB * D, AGG_WORDS)])
            return 0

        lax.fori_loop(0, NPASS, pass_body, 0)

    return agg_kernel(binned, ovf, ovfcnt, feat, coeffr)


def _mm_body(agg_ref, feat_ref, wb_ref, ws_ref, bias_ref, out_ref):
    acc = jnp.dot(agg_ref[...], wb_ref[...], preferred_element_type=jnp.float32)
    acc += jnp.dot(feat_ref[...], ws_ref[...], preferred_element_type=jnp.float32)
    out_ref[...] = acc + bias_ref[...]


def _project(agg_flat, feat, W, bias):
    n = feat.shape[0]
    wb = W[:NB].reshape(NB * D, D)
    ws = W[NB]
    return pl.pallas_call(
        _mm_body,
        grid=(n // ROW_TILE,),
        in_specs=[
            pl.BlockSpec((ROW_TILE, NB * D), lambda i: (i, 0)),
            pl.BlockSpec((ROW_TILE, D), lambda i: (i, 0)),
            pl.BlockSpec((NB * D, D), lambda i: (0, 0)),
            pl.BlockSpec((D, D), lambda i: (0, 0)),
            pl.BlockSpec((D,), lambda i: (0,)),
        ],
        out_specs=pl.BlockSpec((ROW_TILE, D), lambda i: (i, 0)),
        out_shape=jax.ShapeDtypeStruct((n, D), jnp.float32),
    )(agg_flat, feat, wb, ws, bias)


def kernel(feat, edge_index, etypes, W, coeff, bias):
    src = edge_index[0]
    dst = edge_index[1]
    packed = src | (etypes << 14) | (dst << 18)
    binned, ovf, ovfcnt = _sc_bin(packed)
    aggf = _sc_aggregate(binned, ovf, ovfcnt, feat, coeff.reshape(-1))
    agg = aggf.reshape(NRANGE * NLOC, NB * D)[:N_NODES]
    return _project(agg, feat, W, bias)


# speedup vs baseline: 57.8706x; 1.0335x over previous
import functools

import jax
import jax.numpy as jnp
from jax import lax
from jax.experimental import pallas as pl
from jax.experimental.pallas import tpu as pltpu
from jax.experimental.pallas import tpu_sc as plsc

N_NODES = 10000
E_EDGES = 320000
D = 128
NB = 4
NR = 16
NC = 2
NS = 16
NW = NC * NS
L = 16

NPASS = 5
NRANGE = NW * NPASS
NLOC = 64
NBINS = NRANGE
AGG_WORDS = NLOC * NB * D

E_SLAB = E_EDGES // NW
EC = 2000
CCAP = 128
OVF_CAP = 10240
OG = EC // L

GB = 128
CAP = 8192

ROW_TILE = 400

_MESH = dict(core_axis_name="c", subcore_axis_name="s")


def _sc_bin(packed):
    @functools.partial(
        pl.kernel,
        out_type=(
            jax.ShapeDtypeStruct((NW * NBINS * CCAP,), jnp.int32),
            jax.ShapeDtypeStruct((NW * OVF_CAP,), jnp.int32),
            jax.ShapeDtypeStruct((NW * L,), jnp.int32),
        ),
        mesh=plsc.VectorSubcoreMesh(**_MESH),
        scratch_types=[
            pltpu.VMEM((NBINS * CCAP,), jnp.int32),
            pltpu.VMEM((NBINS + L,), jnp.int32),
            pltpu.VMEM((OVF_CAP + L,), jnp.int32),
            pltpu.VMEM((L,), jnp.int32),
            pltpu.VMEM((EC,), jnp.int32),
        ],
        compiler_params=pltpu.CompilerParams(needs_layout_passes=False),
    )
    def bin_kernel(packed_hbm, binned_hbm, ovf_hbm, ovfcnt_hbm,
                   cells, cnt, ovfb, vtmp, pbuf):
        cidx = lax.axis_index("c")
        sidx = lax.axis_index("s")
        wid = sidx * NC + cidx

        sent = jnp.full((L,), -1, jnp.int32)
        zv = jnp.zeros((L,), jnp.int32)

        @plsc.parallel_loop(0, NBINS * CCAP // L, unroll=8)
        def _fill(i):
            cells[pl.ds(i * L, L)] = sent

        @plsc.parallel_loop(0, (NBINS + L) // L, unroll=1)
        def _zcnt(i):
            cnt[pl.ds(i * L, L)] = zv

        def wbody(g, ocur):
            v = pbuf[pl.ds(g * L, L)]
            bins = (v >> 24) & 0xFF
            rank, last = plsc.scan_count(bins)
            cvals = plsc.load_gather(cnt, [bins])
            pos = cvals + rank - 1
            ok = pos < CCAP
            plsc.store_scatter(cells, [bins * CCAP + pos], v, mask=ok)
            plsc.store_scatter(cnt, [bins], cvals + rank, mask=last)
            movf = jnp.logical_not(ok)
            opos = ocur + plsc.cumsum(movf.astype(jnp.int32)) - 1
            plsc.store_scatter(ovfb, [opos], v, mask=movf)
            return ocur + plsc.all_reduce_population_count(movf)

        ocur = jnp.zeros((L,), jnp.int32)
        for c in range(E_SLAB // EC):
            pltpu.sync_copy(
                packed_hbm.at[pl.ds(wid * E_SLAB + c * EC, EC)], pbuf)
            ocur = lax.fori_loop(0, EC // L, wbody, ocur)

        plsc.store_scatter(ovfb, [ocur + lax.iota(jnp.int32, L)], sent)
        vtmp[...] = ocur
        pltpu.sync_copy(vtmp, ovfcnt_hbm.at[pl.ds(wid * L, L)])
        pltpu.sync_copy(cells, binned_hbm.at[pl.ds(wid * NBINS * CCAP, NBINS * CCAP)])
        pltpu.sync_copy(ovfb.at[pl.ds(0, OVF_CAP)],
                        ovf_hbm.at[pl.ds(wid * OVF_CAP, OVF_CAP)])

    return bin_kernel(packed)


def _sc_aggregate(binned, ovf, ovfcnt, feat, coeffr):
    @functools.partial(
        pl.kernel,
        out_type=jax.ShapeDtypeStruct((NRANGE * NLOC * NB * D,), jnp.float32),
        mesh=plsc.VectorSubcoreMesh(**_MESH),
        scratch_types=[
            pltpu.VMEM((CAP + L,), jnp.int32),
            pltpu.VMEM((GB,), jnp.int32),
            pltpu.VMEM((GB,), jnp.int32),
            pltpu.VMEM((2 * GB, D), jnp.float32),
            pltpu.VMEM((AGG_WORDS,), jnp.float32),
            pltpu.VMEM((NW * CCAP,), jnp.int32),
            pltpu.VMEM((EC,), jnp.int32),
            pltpu.VMEM((NW * L,), jnp.int32),
            pltpu.VMEM((NR * NB + L,), jnp.float32),
            pltpu.SemaphoreType.DMA,
            pltpu.SemaphoreType.DMA,
            pltpu.SemaphoreType.DMA,
        ],
        compiler_params=pltpu.CompilerParams(needs_layout_passes=False),
    )
    def agg_kernel(binned_hbm, ovf_hbm, ovfcnt_hbm, feat_hbm, coeff_hbm, out_hbm,
                   psel, gidxA, gidxB, rows, agg, cellbuf, pbuf, ocbuf, coeffv,
                   csem, gsemA, gsemB):
        cidx = lax.axis_index("c")
        sidx = lax.axis_index("s")
        wid = sidx * NC + cidx
        pltpu.sync_copy(coeff_hbm, coeffv.at[pl.ds(0, NR * NB)])
        pltpu.sync_copy(ovfcnt_hbm, ocbuf)

        def accumulate(off, count, rbase, base):
            def acc(e):
                pv = psel[pl.ds(off + e, L)][0]
                dl = ((pv >> 18) & 0x3FFF) - base
                et = (pv >> 14) & 0xF
                cv = coeffv[pl.ds(et * NB, L)]
                c0 = cv[0]
                c1 = cv[1]
                c2 = cv[2]
                c3 = cv[3]
                aoff = dl * (NB * D)
                fs = [rows[rbase + e, pl.ds(q * L, L)] for q in range(D // L)]
                for q in range(D // L):
                    f = fs[q]
                    plsc.addupdate(agg.at[pl.ds(aoff + 0 * D + q * L, L)], f * c0)
                    plsc.addupdate(agg.at[pl.ds(aoff + 1 * D + q * L, L)], f * c1)
                    plsc.addupdate(agg.at[pl.ds(aoff + 2 * D + q * L, L)], f * c2)
                    plsc.addupdate(agg.at[pl.ds(aoff + 3 * D + q * L, L)], f * c3)
            unroll = 2 if isinstance(count, int) else 1
            plsc.parallel_loop(0, count, unroll=unroll)(acc)

        def pass_body(p, _carry):
            rid = p * NW + wid
            base = rid * NLOC

            zvf = jnp.zeros((L,), jnp.float32)

            @plsc.parallel_loop(0, AGG_WORDS // L, unroll=8)
            def _zero(i):
                agg[pl.ds(i * L, L)] = zvf

            def cstart(w2, _):
                pltpu.make_async_copy(
                    binned_hbm.at[pl.ds(w2 * NBINS * CCAP + rid * CCAP, CCAP)],
                    cellbuf.at[pl.ds(w2 * CCAP, CCAP)], csem).start()
                return 0
            lax.fori_loop(0, NW, cstart, 0)

            def cwait(w2, _):
                pltpu.make_async_copy(
                    binned_hbm.at[pl.ds(rid * CCAP, CCAP)],
                    cellbuf.at[pl.ds(0, CCAP)], csem).wait()
                return 0
            lax.fori_loop(0, NW, cwait, 0)

            def build_idx(gidx, off):
                for q in range(GB // L):
                    pv = psel[pl.ds(off + q * L, L)]
                    gidx[pl.ds(q * L, L)] = pv & 0x3FFF

            rowsA = rows.at[pl.ds(0, GB)]
            rowsB = rows.at[pl.ds(GB, GB)]

            def flush_batch(np_):
                off = pl.multiple_of(np_ & (CAP - 1), GB)
                build_idx(gidxA, off)
                gd = pltpu.make_async_copy(feat_hbm.at[gidxA], rowsA, gsemA)
                gd.start()
                gd.wait()
                accumulate(off, GB, 0, base)
                return np_ + GB

            def scan_g(buf, g, cv):
                v = buf[pl.ds(g * L, L)]
                dstf = (v >> 18) & 0x3FFF
                m = (dstf >= base) & (dstf < base + NLOC)
                mi = m.astype(jnp.int32)
                pos = cv + plsc.cumsum(mi) - 1
                plsc.store_scatter(psel, [pos & (CAP - 1)], v, mask=m)
                return cv + plsc.all_reduce_population_count(m)

            cur_vec = plsc.parallel_loop(
                0, NW * CCAP // L, unroll=5,
                carry=jnp.zeros((L,), jnp.int32))(
                    lambda g, cv: scan_g(cellbuf, g, cv))

            curs = jnp.max(cur_vec)
            nbat = curs // GB

            @pl.when(nbat > 0)
            def _():
                build_idx(gidxA, 0)
                pltpu.make_async_copy(feat_hbm.at[gidxA], rowsA, gsemA).start()

            def ring_batch(b, _):
                offn = (b + 1) * GB

                @pl.when((b & 1) == 0)
                def _():
                    @pl.when(b + 1 < nbat)
                    def _():
                        build_idx(gidxB, offn)
                        pltpu.make_async_copy(
                            feat_hbm.at[gidxB], rowsB, gsemB).start()
                    pltpu.make_async_copy(feat_hbm.at[gidxA], rowsA, gsemA).wait()
                    accumulate(b * GB, GB, 0, base)

                @pl.when((b & 1) == 1)
                def _():
                    @pl.when(b + 1 < nbat)
                    def _():
                        build_idx(gidxA, offn)
                        pltpu.make_async_copy(
                            feat_hbm.at[gidxA], rowsA, gsemA).start()
                    pltpu.make_async_copy(feat_hbm.at[gidxB], rowsB, gsemB).wait()
                    accumulate(b * GB, GB, GB, base)
                return 0

            lax.fori_loop(0, nbat, ring_batch, 0)
            nproc = nbat * GB

            def ovf_w(w2, carry):
                cur_vec2, nproc2 = carry
                cw = ocbuf[pl.ds(w2 * L, L)][0]
                ngroups = (cw + L - 1) >> 4
                nchunks = (ngroups + OG - 1) // OG

                def ochunk(c, carry3):
                    cur_vec3, nproc3 = carry3
                    pltpu.sync_copy(
                        ovf_hbm.at[pl.ds(w2 * OVF_CAP + c * EC, EC)], pbuf)
                    gs = jnp.minimum(ngroups - c * OG, OG)
                    cur_vec3 = lax.fori_loop(
                        0, gs, lambda g, cv: scan_g(pbuf, g, cv), cur_vec3)
                    curs3 = jnp.max(cur_vec3)
                    nproc3 = lax.while_loop(
                        lambda np_: np_ + GB <= curs3, flush_batch, nproc3)
                    return cur_vec3, nproc3

                return lax.fori_loop(0, nchunks, ochunk, (cur_vec2, nproc2))

            cur_vec, nproc = lax.fori_loop(0, NW, ovf_w, (cur_vec, nproc))

            pending = jnp.max(cur_vec) - nproc
            off = pl.multiple_of(nproc & (CAP - 1), GB)

            @pl.when(pending > 0)
            def _():
                zidx = jnp.zeros((L,), jnp.int32)
                for q in range(GB // L):
                    pv = psel[pl.ds(off + q * L, L)]
                    lane = lax.iota(jnp.int32, L) + q * L
                    gidxA[pl.ds(q * L, L)] = jnp.where(lane < pending, pv & 0x3FFF, zidx)
                gd = pltpu.make_async_copy(feat_hbm.at[gidxA], rowsA, gsemA)
                gd.start()
                gd.wait()
                accumulate(off, pending, 0, base)

            pltpu.sync_copy(agg, out_hbm.at[pl.ds(base * NB * D, AGG_WORDS)])
            return 0

        lax.fori_loop(0, NPASS, pass_body, 0)

    return agg_kernel(binned, ovf, ovfcnt, feat, coeffr)


def _mm_body(agg_ref, feat_ref, wb_ref, ws_ref, bias_ref, out_ref):
    acc = jnp.dot(agg_ref[...], wb_ref[...], preferred_element_type=jnp.float32)
    acc += jnp.dot(feat_ref[...], ws_ref[...], preferred_element_type=jnp.float32)
    out_ref[...] = acc + bias_ref[...]


def _project(agg_flat, feat, W, bias):
    n = feat.shape[0]
    wb = W[:NB].reshape(NB * D, D)
    ws = W[NB]
    return pl.pallas_call(
        _mm_body,
        grid=(n // ROW_TILE,),
        in_specs=[
            pl.BlockSpec((ROW_TILE, NB * D), lambda i: (i, 0)),
            pl.BlockSpec((ROW_TILE, D), lambda i: (i, 0)),
            pl.BlockSpec((NB * D, D), lambda i: (0, 0)),
            pl.BlockSpec((D, D), lambda i: (0, 0)),
            pl.BlockSpec((D,), lambda i: (0,)),
        ],
        out_specs=pl.BlockSpec((ROW_TILE, D), lambda i: (i, 0)),
        out_shape=jax.ShapeDtypeStruct((n, D), jnp.float32),
    )(agg_flat, feat, wb, ws, bias)


def kernel(feat, edge_index, etypes, W, coeff, bias):
    src = edge_index[0]
    dst = edge_index[1]
    packed = src | (etypes << 14) | (dst << 18)
    binned, ovf, ovfcnt = _sc_bin(packed)
    aggf = _sc_aggregate(binned, ovf, ovfcnt, feat, coeff.reshape(-1))
    agg = aggf.reshape(NRANGE * NLOC, NB * D)[:N_NODES]
    return _project(agg, feat, W, bias)
